# baseline (speedup 1.0000x reference)
import functools

import jax
import jax.numpy as jnp
from jax import lax
from jax.experimental import pallas as pl
from jax.experimental.pallas import tpu as pltpu
from jax.experimental.pallas import tpu_sc as plsc

IDX_H2 = 10

NC = 2
NS = 16
L = 16
NW = NC * NS

_LOG2_10 = 3.321928094887362
_S1 = 127.0 / (13.0 * _LOG2_10)
_S0 = -10.0 * _LOG2_10 * _S1


def _make_sc_call(B, K, W):
    chunk = B // NW
    steps = chunk // L
    mesh = plsc.VectorSubcoreMesh(core_axis_name="c", subcore_axis_name="s",
                                  num_cores=NC, num_subcores=NS)

    @functools.partial(
        pl.kernel,
        out_type=jax.ShapeDtypeStruct((B,), jnp.float32),
        mesh=mesh,
        compiler_params=pltpu.CompilerParams(needs_layout_passes=False),
        scratch_types=[
            pltpu.VMEM((chunk,), jnp.float32),
            pltpu.VMEM((chunk,), jnp.float32),
            pltpu.VMEM((chunk,), jnp.int32),
            pltpu.VMEM((K,), jnp.float32),
            pltpu.VMEM((K,), jnp.float32),
            pltpu.VMEM((L,), jnp.float32),
            pltpu.SemaphoreType.DMA,
        ],
    )
    def sc_call(av_hbm, yf_hbm, xt_hbm, fac_hbm, cvec_hbm, out_hbm,
                av_v, oy_v, idx_v, xt_v, fac_v, c_v, sem):
        wid = lax.axis_index("s") * NC + lax.axis_index("c")
        base = wid * chunk
        iota = lax.iota(jnp.int32, L)
        iw = iota * W
        off0 = (base + 0) * W + IDX_H2

        def gen(i):
            idx_v[pl.ds(i * L, L)] = iw + (off0 + i * (L * W))

        plsc.parallel_loop(0, steps, 1, unroll=8)(gen)
        av_copy = pltpu.async_copy(av_hbm.at[pl.ds(base, chunk)], av_v, sem)
        y_copy = pltpu.async_copy(yf_hbm.at[idx_v], oy_v, sem)
        pltpu.sync_copy(xt_hbm, xt_v)
        pltpu.sync_copy(fac_hbm, fac_v)
        pltpu.sync_copy(cvec_hbm, c_v)
        av_copy.wait()
        y_copy.wait()
        c = c_v[...]

        def step(i):
            sl = pl.ds(i * L, L)
            q = (av_v[sl] * c) * oy_v[sl]
            bits = lax.bitcast_convert_type(q, jnp.int32)
            zf = bits.astype(jnp.float32) * (1.0 / (1 << 23)) - 127.0
            idx_f = jnp.clip(zf * _S1 + _S0, 0.0, float(K - 3))
            j = idx_f.astype(jnp.int32)
            xm = plsc.load_gather(xt_v, [j + 1])
            up = q >= xm
            i0 = jnp.where(up, j + 1, j)
            xo = plsc.load_gather(xt_v, [jnp.where(up, j + 2, j)])
            x0 = jnp.where(up, xm, xo)
            x1 = jnp.where(up, xo, xm)
            f0 = plsc.load_gather(fac_v, [i0])
            f1 = plsc.load_gather(fac_v, [i0 + 1])
            t = jnp.clip((q - x0) / (x1 - x0), 0.0, 1.0)
            oy_v[sl] = f0 + (f1 - f0) * t

        plsc.parallel_loop(0, steps, 1, unroll=8)(step)
        pltpu.sync_copy(oy_v, out_hbm.at[pl.ds(base, chunk)])

    return sc_call


def kernel(Av, params_reac, y_in, x_H2, factor, den_Av_ratio_0):
    B = Av.shape[0]
    K = x_H2.shape[0]
    W = y_in.shape[1]
    av = Av.reshape(B)
    yf = y_in.reshape(B * W)
    fac = factor.reshape(K)
    cvec = jnp.full((L,), den_Av_ratio_0, dtype=jnp.float32)
    out = _make_sc_call(B, K, W)(av, yf, x_H2, fac, cvec)
    return out.reshape(B, 1)


# speedup vs baseline: 13.9821x; 13.9821x over previous
import functools

import jax
import jax.numpy as jnp
from jax import lax
from jax.experimental import pallas as pl
from jax.experimental.pallas import tpu as pltpu
from jax.experimental.pallas import tpu_sc as plsc

IDX_H2 = 10

NC = 2
NS = 16
L = 16
NW = NC * NS
NSUB = 8

_LOG2_10 = 3.321928094887362
_S1 = 127.0 / (13.0 * _LOG2_10)
_S0 = -10.0 * _LOG2_10 * _S1


def _make_sc_call(B, K):
    chunk = B // NW
    sub = chunk // NSUB
    steps = sub // L
    mesh = plsc.VectorSubcoreMesh(core_axis_name="c", subcore_axis_name="s",
                                  num_cores=NC, num_subcores=NS)

    @functools.partial(
        pl.kernel,
        out_type=jax.ShapeDtypeStruct((B,), jnp.float32),
        mesh=mesh,
        compiler_params=pltpu.CompilerParams(needs_layout_passes=False),
        scratch_types=[
            pltpu.VMEM((sub,), jnp.float32),
            pltpu.VMEM((sub,), jnp.float32),
            pltpu.VMEM((sub,), jnp.float32),
            pltpu.VMEM((sub,), jnp.float32),
            pltpu.VMEM((sub,), jnp.float32),
            pltpu.VMEM((sub,), jnp.float32),
            pltpu.VMEM((K,), jnp.float32),
            pltpu.VMEM((K,), jnp.float32),
            pltpu.VMEM((L,), jnp.float32),
            pltpu.SemaphoreType.DMA,
            pltpu.SemaphoreType.DMA,
            pltpu.SemaphoreType.DMA,
            pltpu.SemaphoreType.DMA,
            pltpu.SemaphoreType.DMA,
        ],
    )
    def sc_call(av_hbm, yc_hbm, xt_hbm, fac_hbm, cvec_hbm, out_hbm,
                av0, av1, yc0, yc1, ot0, ot1, xt_v, fac_v, c_v,
                sem_in0, sem_in1, sem_out0, sem_out1, sem_t):
        wid = lax.axis_index("s") * NC + lax.axis_index("c")
        base = wid * chunk
        av_s = (av0, av1)
        yc_s = (yc0, yc1)
        out_s = (ot0, ot1)
        sems_in = (sem_in0, sem_in1)
        sems_out = (sem_out0, sem_out1)

        t0 = pltpu.async_copy(xt_hbm, xt_v, sem_t)
        t1 = pltpu.async_copy(fac_hbm, fac_v, sem_t)
        t2 = pltpu.async_copy(cvec_hbm, c_v, sem_t)

        def start_in(g):
            s = g % 2
            lo = base + g * sub
            a = pltpu.async_copy(av_hbm.at[pl.ds(lo, sub)], av_s[s], sems_in[s])
            y = pltpu.async_copy(yc_hbm.at[pl.ds(lo, sub)], yc_s[s], sems_in[s])
            return (a, y)

        pend_in = {0: start_in(0)}
        pend_out = {}
        t0.wait(); t1.wait(); t2.wait()
        c = c_v[...]

        for g in range(NSUB):
            s = g % 2
            if g + 1 < NSUB:
                pend_in[g + 1] = start_in(g + 1)
            for d in pend_in.pop(g):
                d.wait()
            if g - 2 in pend_out:
                pend_out.pop(g - 2).wait()
            avb, yb, ob = av_s[s], yc_s[s], out_s[s]

            def step(i, avb=avb, yb=yb, ob=ob):
                sl = pl.ds(i * L, L)
                q = (avb[sl] * c) * yb[sl]
                bits = lax.bitcast_convert_type(q, jnp.int32)
                zf = bits.astype(jnp.float32) * (1.0 / (1 << 23)) - 127.0
                idx_f = jnp.clip(zf * _S1 + _S0, 0.0, float(K - 3))
                j = idx_f.astype(jnp.int32)
                xm = plsc.load_gather(xt_v, [j + 1])
                up = q >= xm
                i0 = jnp.where(up, j + 1, j)
                xo = plsc.load_gather(xt_v, [jnp.where(up, j + 2, j)])
                x0 = jnp.where(up, xm, xo)
                x1 = jnp.where(up, xo, xm)
                f0 = plsc.load_gather(fac_v, [i0])
                f1 = plsc.load_gather(fac_v, [i0 + 1])
                t = jnp.clip((q - x0) / (x1 - x0), 0.0, 1.0)
                ob[sl] = f0 + (f1 - f0) * t

            plsc.parallel_loop(0, steps, 1, unroll=8)(step)
            pend_out[g] = pltpu.async_copy(
                ob, out_hbm.at[pl.ds(base + g * sub, sub)], sems_out[s])
        for g in sorted(pend_out):
            pend_out.pop(g).wait()

    return sc_call


def kernel(Av, params_reac, y_in, x_H2, factor, den_Av_ratio_0):
    B = Av.shape[0]
    K = x_H2.shape[0]
    av = Av.reshape(B)
    yc = y_in[:, IDX_H2]
    fac = factor.reshape(K)
    cvec = jnp.full((L,), den_Av_ratio_0, dtype=jnp.float32)
    out = _make_sc_call(B, K)(av, yc, x_H2, fac, cvec)
    return out.reshape(B, 1)


# speedup vs baseline: 14.4118x; 1.0307x over previous
import functools

import jax
import jax.numpy as jnp
from jax import lax
from jax.experimental import pallas as pl
from jax.experimental.pallas import tpu as pltpu
from jax.experimental.pallas import tpu_sc as plsc

IDX_H2 = 10

NC = 2
NS = 16
L = 16
NW = NC * NS
NSUB = 8

_LOG2_10 = 3.321928094887362
_S1 = 127.0 / (13.0 * _LOG2_10)
_A = _S1 / float(1 << 23)


def _make_sc_call(B, K):
    chunk = B // NW
    sub = chunk // NSUB
    steps = sub // L
    mesh = plsc.VectorSubcoreMesh(core_axis_name="c", subcore_axis_name="s",
                                  num_cores=NC, num_subcores=NS)

    @functools.partial(
        pl.kernel,
        out_type=jax.ShapeDtypeStruct((B,), jnp.float32),
        mesh=mesh,
        compiler_params=pltpu.CompilerParams(needs_layout_passes=False),
        scratch_types=[
            pltpu.VMEM((sub,), jnp.float32),
            pltpu.VMEM((sub,), jnp.float32),
            pltpu.VMEM((sub,), jnp.float32),
            pltpu.VMEM((sub,), jnp.float32),
            pltpu.VMEM((sub,), jnp.float32),
            pltpu.VMEM((sub,), jnp.float32),
            pltpu.VMEM((K,), jnp.float32),
            pltpu.VMEM((K,), jnp.float32),
            pltpu.VMEM((K,), jnp.float32),
            pltpu.VMEM((K,), jnp.float32),
            pltpu.VMEM((L,), jnp.float32),
            pltpu.SemaphoreType.DMA,
            pltpu.SemaphoreType.DMA,
            pltpu.SemaphoreType.DMA,
            pltpu.SemaphoreType.DMA,
            pltpu.SemaphoreType.DMA,
        ],
    )
    def sc_call(av_hbm, yc_hbm, xs_hbm, rdx_hbm, w_hbm, fac_hbm, bc_hbm,
                out_hbm,
                av0, av1, yc0, yc1, ot0, ot1, xs_v, rdx_v, w_v, fac_v, bc_v,
                sem_in0, sem_in1, sem_out0, sem_out1, sem_t):
        wid = lax.axis_index("s") * NC + lax.axis_index("c")
        base = wid * chunk
        av_s = (av0, av1)
        yc_s = (yc0, yc1)
        out_s = (ot0, ot1)
        sems_in = (sem_in0, sem_in1)
        sems_out = (sem_out0, sem_out1)

        tcopies = [
            pltpu.async_copy(xs_hbm, xs_v, sem_t),
            pltpu.async_copy(rdx_hbm, rdx_v, sem_t),
            pltpu.async_copy(w_hbm, w_v, sem_t),
            pltpu.async_copy(fac_hbm, fac_v, sem_t),
            pltpu.async_copy(bc_hbm, bc_v, sem_t),
        ]

        def start_in(g):
            s = g % 2
            lo = base + g * sub
            a = pltpu.async_copy(av_hbm.at[pl.ds(lo, sub)], av_s[s], sems_in[s])
            y = pltpu.async_copy(yc_hbm.at[pl.ds(lo, sub)], yc_s[s], sems_in[s])
            return (a, y)

        pend_in = {0: start_in(0)}
        pend_out = {}
        for d in tcopies:
            d.wait()
        bc = bc_v[...]

        for g in range(NSUB):
            s = g % 2
            if g + 1 < NSUB:
                pend_in[g + 1] = start_in(g + 1)
            for d in pend_in.pop(g):
                d.wait()
            if g - 2 in pend_out:
                pend_out.pop(g - 2).wait()
            avb, yb, ob = av_s[s], yc_s[s], out_s[s]

            def step(i, avb=avb, yb=yb, ob=ob):
                sl = pl.ds(i * L, L)
                q = avb[sl] * yb[sl]
                bits = lax.bitcast_convert_type(q, jnp.int32)
                idx_f = jnp.clip(bits.astype(jnp.float32) * _A + bc,
                                 0.0, float(K - 3))
                j = idx_f.astype(jnp.int32)
                jp = j + 1
                xm = plsc.load_gather(xs_v, [jp])
                i0 = jnp.where(q >= xm, jp, j)
                rdx0 = plsc.load_gather(rdx_v, [i0])
                w0 = plsc.load_gather(w_v, [i0])
                f0 = plsc.load_gather(fac_v, [i0])
                f1 = plsc.load_gather(fac_v, [i0 + 1])
                t = jnp.clip(q * rdx0 + w0, 0.0, 1.0)
                ob[sl] = f0 + (f1 - f0) * t

            plsc.parallel_loop(0, steps, 1, unroll=8)(step)
            pend_out[g] = pltpu.async_copy(
                ob, out_hbm.at[pl.ds(base + g * sub, sub)], sems_out[s])
        for g in sorted(pend_out):
            pend_out.pop(g).wait()

    return sc_call


def kernel(Av, params_reac, y_in, x_H2, factor, den_Av_ratio_0):
    B = Av.shape[0]
    K = x_H2.shape[0]
    av = Av.reshape(B)
    yc = y_in[:, IDX_H2]
    fac = factor.reshape(K)
    c = den_Av_ratio_0.astype(jnp.float32)
    xs = x_H2 / c
    dx = xs[1:] - xs[:-1]
    rdx = jnp.concatenate([1.0 / dx, jnp.zeros((1,), jnp.float32)])
    w = -xs * rdx
    bc = jnp.full(
        (L,),
        -_S1 * (127.0 + jnp.log2(xs[0].astype(jnp.float64))),
        dtype=jnp.float32,
    )
    out = _make_sc_call(B, K)(av, yc, xs, rdx, w, fac, bc)
    return out.reshape(B, 1)
